# Initial kernel scaffold; baseline (speedup 1.0000x reference)
import jax
import jax.numpy as jnp
from jax.experimental import pallas as pl


def kernel(x, edge_index, edge_attr, x_pos, filt_w1, filt_b1, filt_w2, filt_b2, lin1_w, lin2_w, lin2_b, lin3_w, lin3_b, emlp_w, emlp_b):
    raise NotImplementedError("write your pallas kernel here")


# speedup vs baseline: 2.2094x; 2.2094x over previous
import functools
import math

import jax
import jax.numpy as jnp
from jax import lax
from jax.experimental import pallas as pl
from jax.experimental.pallas import tpu as pltpu
from jax.experimental.pallas import tpu_sc as plsc

N = 10000
E = 160000
D = 128
NF = 256
CUTOFF = 10.0
LOG2 = math.log(2.0)

NC = 2
NS = 16
NW = NC * NS
CH = 128
NCHUNK = E // CH
LANES = 16
ZCH = 128
NZCHUNKS = N // ZCH
ZREM = N - NZCHUNKS * ZCH

NB = 2000
EB = 2000


def _ssp(v):
    return jax.nn.softplus(v) - LOG2


def _mesh():
    return plsc.VectorSubcoreMesh(core_axis_name="c", subcore_axis_name="s",
                                  num_cores=NC, num_subcores=NS)


def _dist2_body(xs_hbm, ys_hbm, zs_hbm, row_hbm, col_hbm, d2_hbm,
                xs, ys, zs, idxr, idxc, obuf):
    wid = lax.axis_index("s") * NC + lax.axis_index("c")
    pltpu.sync_copy(xs_hbm, xs)
    pltpu.sync_copy(ys_hbm, ys)
    pltpu.sync_copy(zs_hbm, zs)
    nchunks = (NCHUNK - 1 - wid) // NW + 1

    def body(i, carry):
        c = wid + i * NW
        base = c * CH
        pltpu.sync_copy(row_hbm.at[pl.ds(base, CH)], idxr)
        pltpu.sync_copy(col_hbm.at[pl.ds(base, CH)], idxc)
        for g in range(CH // LANES):
            sl = pl.ds(g * LANES, LANES)
            ir = idxr[sl]
            ic = idxc[sl]
            dx = plsc.load_gather(xs, [ir]) - plsc.load_gather(xs, [ic])
            dy = plsc.load_gather(ys, [ir]) - plsc.load_gather(ys, [ic])
            dz = plsc.load_gather(zs, [ir]) - plsc.load_gather(zs, [ic])
            obuf[sl] = (dx * dx + dy * dy) + dz * dz
        pltpu.sync_copy(obuf, d2_hbm.at[pl.ds(base, CH)])
        return carry

    lax.fori_loop(0, nchunks, body, 0)


@functools.lru_cache(maxsize=None)
def _dist2_kernel():
    return pl.kernel(
        _dist2_body,
        out_type=jax.ShapeDtypeStruct((E,), jnp.float32),
        mesh=_mesh(),
        compiler_params=pltpu.CompilerParams(needs_layout_passes=False),
        scratch_types=[
            pltpu.VMEM((N,), jnp.float32),
            pltpu.VMEM((N,), jnp.float32),
            pltpu.VMEM((N,), jnp.float32),
            pltpu.VMEM((CH,), jnp.int32),
            pltpu.VMEM((CH,), jnp.int32),
            pltpu.VMEM((CH,), jnp.float32),
        ],
    )


def _scatter_body(hcat, wcat, row_hbm, col_hbm, zeros_hbm, agg_hbm,
                  idxr, idxc, idxa, hbuf, wbuf, acc, sem):
    cid = lax.axis_index("c")
    sid = lax.axis_index("s")
    coff_h = cid * N
    nz = (NZCHUNKS - 1 - sid) // NS + 1

    def zbody(i, carry):
        r = (sid + i * NS) * ZCH
        pltpu.sync_copy(zeros_hbm, acc.at[pl.ds(r, ZCH)])
        return carry

    lax.fori_loop(0, nz, zbody, 0)

    @pl.when(sid == 0)
    def _():
        pltpu.sync_copy(zeros_hbm.at[pl.ds(0, ZREM)],
                        acc.at[pl.ds(NZCHUNKS * ZCH, ZREM)])

    plsc.subcore_barrier()

    nchunks = (NCHUNK - 1 - sid) // NS + 1

    def body(i, carry):
        c = sid + i * NS
        base = c * CH
        pltpu.sync_copy(row_hbm.at[pl.ds(base, CH)], idxr)
        pltpu.sync_copy(col_hbm.at[pl.ds(base, CH)], idxc)
        for g in range(CH // LANES):
            sl = pl.ds(g * LANES, LANES)
            idxa[sl] = idxc[sl] + coff_h
        pltpu.async_copy(hcat.at[idxa], hbuf, sem).wait()
        pltpu.sync_copy(wcat.at[pl.ds(cid * E + base, CH)], wbuf)

        def mul_body(e, carry2):
            for g in range(D // LANES):
                sl = pl.ds(g * LANES, LANES)
                hbuf[e, sl] = hbuf[e, sl] * wbuf[e, sl]
            return carry2

        lax.fori_loop(0, CH, mul_body, 0)
        pltpu.sync_copy(hbuf, acc.at[idxr], add=True)
        return carry

    lax.fori_loop(0, nchunks, body, 0)
    plsc.subcore_barrier()

    def obody(i, carry):
        r = (sid + i * NS) * ZCH
        pltpu.sync_copy(acc.at[pl.ds(r, ZCH)],
                        agg_hbm.at[pl.ds(coff_h + r, ZCH)])
        return carry

    lax.fori_loop(0, nz, obody, 0)

    @pl.when(sid == 0)
    def _():
        r = NZCHUNKS * ZCH
        pltpu.sync_copy(acc.at[pl.ds(r, ZREM)],
                        agg_hbm.at[pl.ds(coff_h + r, ZREM)])


@functools.lru_cache(maxsize=None)
def _scatter_kernel():
    return pl.kernel(
        _scatter_body,
        out_type=jax.ShapeDtypeStruct((2 * N, D), jnp.float32),
        mesh=_mesh(),
        compiler_params=pltpu.CompilerParams(needs_layout_passes=False),
        scratch_types=[
            pltpu.VMEM((CH,), jnp.int32),
            pltpu.VMEM((CH,), jnp.int32),
            pltpu.VMEM((CH,), jnp.int32),
            pltpu.VMEM((CH, D), jnp.float32),
            pltpu.VMEM((CH, D), jnp.float32),
            pltpu.VMEM_SHARED((N, D), jnp.float32),
            pltpu.SemaphoreType.DMA,
        ],
    )


def _edge_gather_body(xnew, row_hbm, col_hbm, g_hbm,
                      idxr, idxc, rbuf, cbuf, sem1, sem2):
    wid = lax.axis_index("s") * NC + lax.axis_index("c")
    nchunks = (NCHUNK - 1 - wid) // NW + 1

    def body(i, carry):
        c = wid + i * NW
        base = c * CH
        pltpu.sync_copy(row_hbm.at[pl.ds(base, CH)], idxr)
        pltpu.sync_copy(col_hbm.at[pl.ds(base, CH)], idxc)
        cp1 = pltpu.async_copy(xnew.at[idxr], rbuf, sem1)
        cp2 = pltpu.async_copy(xnew.at[idxc], cbuf, sem2)
        cp1.wait()
        cp2.wait()

        def add_body(e, carry2):
            for g in range(D // LANES):
                sl = pl.ds(g * LANES, LANES)
                rbuf[e, sl] = rbuf[e, sl] + cbuf[e, sl]
            return carry2

        lax.fori_loop(0, CH, add_body, 0)
        pltpu.sync_copy(rbuf, g_hbm.at[pl.ds(base, CH)])
        return carry

    lax.fori_loop(0, nchunks, body, 0)


@functools.lru_cache(maxsize=None)
def _edge_gather_kernel():
    return pl.kernel(
        _edge_gather_body,
        out_type=jax.ShapeDtypeStruct((E, D), jnp.float32),
        mesh=_mesh(),
        compiler_params=pltpu.CompilerParams(needs_layout_passes=False),
        scratch_types=[
            pltpu.VMEM((CH,), jnp.int32),
            pltpu.VMEM((CH,), jnp.int32),
            pltpu.VMEM((CH, D), jnp.float32),
            pltpu.VMEM((CH, D), jnp.float32),
            pltpu.SemaphoreType.DMA,
            pltpu.SemaphoreType.DMA,
        ],
    )


def _h_body(x_ref, w_ref, o_ref):
    x = x_ref[...]
    w = w_ref[...]
    o_ref[0] = jnp.dot(x, w[:, :D], preferred_element_type=jnp.float32)
    o_ref[1] = jnp.dot(x, w[:, D:], preferred_element_type=jnp.float32)


def _h_call(x, lin1_w):
    return pl.pallas_call(
        _h_body,
        grid=(N // NB,),
        in_specs=[
            pl.BlockSpec((NB, D), lambda i: (i, 0)),
            pl.BlockSpec((D, NF), lambda i: (0, 0)),
        ],
        out_specs=pl.BlockSpec((2, NB, D), lambda i: (0, i, 0)),
        out_shape=jax.ShapeDtypeStruct((2, N, D), jnp.float32),
    )(x, lin1_w)


def _filt_body(ea_ref, d2_ref, w1_ref, b1_ref, w2_ref, b2_ref, o_ref):
    ea = ea_ref[...]
    hid = _ssp(jnp.dot(ea, w1_ref[...], preferred_element_type=jnp.float32)
               + b1_ref[...])
    d2 = d2_ref[...][:, 0]
    dist = jnp.sqrt(d2 + 1e-12)
    env = 0.5 * (jnp.cos(dist * (math.pi / CUTOFF)) + 1.0)
    env = env * (dist < CUTOFF).astype(jnp.float32)
    w2 = w2_ref[...]
    b2 = b2_ref[...]
    for f in range(2):
        wv = (jnp.dot(hid, w2[:, f * D:(f + 1) * D],
                      preferred_element_type=jnp.float32)
              + b2[:, f * D:(f + 1) * D])
        o_ref[f] = wv * env[:, None]


def _filt_call(edge_attr, d2, filt_w1, filt_b1, filt_w2, filt_b2):
    return pl.pallas_call(
        _filt_body,
        grid=(E // EB,),
        in_specs=[
            pl.BlockSpec((EB, D), lambda i: (i, 0)),
            pl.BlockSpec((EB, 1), lambda i: (i, 0)),
            pl.BlockSpec((D, NF), lambda i: (0, 0)),
            pl.BlockSpec((1, NF), lambda i: (0, 0)),
            pl.BlockSpec((NF, NF), lambda i: (0, 0)),
            pl.BlockSpec((1, NF), lambda i: (0, 0)),
        ],
        out_specs=pl.BlockSpec((2, EB, D), lambda i: (0, i, 0)),
        out_shape=jax.ShapeDtypeStruct((2, E, D), jnp.float32),
    )(edge_attr, d2, filt_w1, filt_b1, filt_w2, filt_b2)


def _node_body(a_ref, x_ref, w2_ref, b2_ref, w3_ref, b3_ref, o_ref):
    w2 = w2_ref[...]
    t = (jnp.dot(a_ref[0], w2[:NF // 2], preferred_element_type=jnp.float32)
         + jnp.dot(a_ref[1], w2[NF // 2:], preferred_element_type=jnp.float32)
         + b2_ref[...])
    t = _ssp(t)
    o = jnp.dot(t, w3_ref[...], preferred_element_type=jnp.float32) + b3_ref[...]
    o_ref[...] = jnp.maximum(o, 0.0) + x_ref[...]


def _node_call(agg2, x, lin2_w, lin2_b, lin3_w, lin3_b):
    return pl.pallas_call(
        _node_body,
        grid=(N // NB,),
        in_specs=[
            pl.BlockSpec((2, NB, D), lambda i: (0, i, 0)),
            pl.BlockSpec((NB, D), lambda i: (i, 0)),
            pl.BlockSpec((NF, D), lambda i: (0, 0)),
            pl.BlockSpec((1, D), lambda i: (0, 0)),
            pl.BlockSpec((D, D), lambda i: (0, 0)),
            pl.BlockSpec((1, D), lambda i: (0, 0)),
        ],
        out_specs=pl.BlockSpec((NB, D), lambda i: (i, 0)),
        out_shape=jax.ShapeDtypeStruct((N, D), jnp.float32),
    )(agg2, x, lin2_w, lin2_b, lin3_w, lin3_b)


def _emlp_body(ea_ref, g_ref, w_ref, b_ref, o_ref):
    ea = ea_ref[...]
    a = jnp.maximum(ea, 0.0)
    b = jnp.maximum(g_ref[...], 0.0)
    w = w_ref[...]
    z = (jnp.dot(a, w[:D], preferred_element_type=jnp.float32)
         + jnp.dot(b, w[D:], preferred_element_type=jnp.float32)
         + b_ref[...])
    o_ref[...] = jnp.tanh(z) + ea


def _emlp_call(edge_attr, g, emlp_w, emlp_b):
    return pl.pallas_call(
        _emlp_body,
        grid=(E // EB,),
        in_specs=[
            pl.BlockSpec((EB, D), lambda i: (i, 0)),
            pl.BlockSpec((EB, D), lambda i: (i, 0)),
            pl.BlockSpec((2 * D, D), lambda i: (0, 0)),
            pl.BlockSpec((1, D), lambda i: (0, 0)),
        ],
        out_specs=pl.BlockSpec((EB, D), lambda i: (i, 0)),
        out_shape=jax.ShapeDtypeStruct((E, D), jnp.float32),
    )(edge_attr, g, emlp_w, emlp_b)


def kernel(x, edge_index, edge_attr, x_pos,
           filt_w1, filt_b1, filt_w2, filt_b2,
           lin1_w, lin2_w, lin2_b, lin3_w, lin3_b,
           emlp_w, emlp_b):
    row = edge_index[0]
    col = edge_index[1]
    xs = x_pos[:, 0]
    ys = x_pos[:, 1]
    zs = x_pos[:, 2]

    d2 = _dist2_kernel()(xs, ys, zs, row, col)
    h2 = _h_call(x, lin1_w)
    wc2 = _filt_call(edge_attr, d2.reshape(E, 1),
                     filt_w1, filt_b1.reshape(1, NF),
                     filt_w2, filt_b2.reshape(1, NF))
    zeros = jnp.zeros((ZCH, D), jnp.float32)
    agg2 = _scatter_kernel()(h2.reshape(2 * N, D), wc2.reshape(2 * E, D),
                             row, col, zeros)
    x_new = _node_call(agg2.reshape(2, N, D), x,
                       lin2_w, lin2_b.reshape(1, D),
                       lin3_w, lin3_b.reshape(1, D))
    g = _edge_gather_kernel()(x_new, row, col)
    e_new = _emlp_call(edge_attr, g, emlp_w, emlp_b.reshape(1, D))
    return (x_new, e_new)
